# Initial kernel scaffold; baseline (speedup 1.0000x reference)
import jax
import jax.numpy as jnp
from jax.experimental import pallas as pl


def kernel(memory, source_nodes, timestamps, n_layers, n_neighbors, time_diffs, W, b):
    raise NotImplementedError("write your pallas kernel here")


# speedup vs baseline: 1.0997x; 1.0997x over previous
import functools
import math

import jax
import jax.numpy as jnp
from jax import lax
from jax.experimental import pallas as pl
from jax.experimental.pallas import tpu as pltpu
from jax.experimental.pallas import tpu_sc as plsc

N_NODES = 1000000
D = 128
B = 16384
L = 16
NC = 2
NS = 16
NW = NC * NS
B_PER_W = B // NW
N_CHUNK = B_PER_W // 128


@functools.lru_cache(maxsize=1)
def _build_kernel():
    mesh = plsc.VectorSubcoreMesh(
        core_axis_name="c", subcore_axis_name="s",
        num_cores=NC, num_subcores=NS)

    @functools.partial(
        pl.kernel,
        out_type=jax.ShapeDtypeStruct((B, D), jnp.float32),
        mesh=mesh,
        scratch_types=[
            pltpu.VMEM((N_CHUNK, 128), jnp.int32),
            pltpu.VMEM((B_PER_W, D), jnp.float32),
            pltpu.VMEM((D,), jnp.float32),
            pltpu.VMEM((D,), jnp.float32),
            pltpu.VMEM((B_PER_W,), jnp.float32),
            pltpu.SemaphoreType.DMA,
        ],
    )
    def time_embed(mem_hbm, idx_hbm, td_hbm, w_hbm, bp_hbm, out_hbm,
                   idx_v, rows_v, w_v, bp_v, td_v, sem):
        wid = lax.axis_index("s") * NC + lax.axis_index("c")
        base = wid * B_PER_W

        pltpu.sync_copy(idx_hbm.at[wid], idx_v)
        pltpu.sync_copy(td_hbm.at[pl.ds(base, B_PER_W)], td_v)
        pltpu.sync_copy(w_hbm, w_v)
        pltpu.sync_copy(bp_hbm, bp_v)

        copies = []
        for k in range(N_CHUNK):
            copies.append(pltpu.async_copy(
                mem_hbm.at[idx_v.at[k]],
                rows_v.at[pl.ds(k * 128, 128)],
                sem,
            ))
        for c in copies:
            c.wait()

        w_regs = [w_v[pl.ds(j * L, L)] for j in range(D // L)]
        bp_regs = [bp_v[pl.ds(j * L, L)] for j in range(D // L)]

        def group_body(g, carry):
            tdv = td_v[pl.ds(g * L, L)]
            for r in range(L):
                i = g * L + r
                td = tdv[r]
                for j in range(D // L):
                    sl = pl.ds(j * L, L)
                    rows_v[i, sl] = rows_v[i, sl] * (td * w_regs[j] + bp_regs[j])
            return carry

        lax.fori_loop(0, B_PER_W // L, group_body, 0, unroll=False)

        pltpu.sync_copy(rows_v, out_hbm.at[pl.ds(base, B_PER_W)])

    return time_embed


def kernel(memory, source_nodes, timestamps, n_layers, n_neighbors,
           time_diffs, W, b):
    idx = source_nodes.astype(jnp.int32).reshape(NW, N_CHUNK, 128)
    w = W[:, 0]
    bp = 1.0 + b
    fn = _build_kernel()
    return fn(memory, idx, time_diffs, w, bp)
